# Initial kernel scaffold; baseline (speedup 1.0000x reference)
import functools

import jax
import jax.numpy as jnp
from jax import lax
from jax.experimental import pallas as pl
from jax.experimental.pallas import tpu as pltpu
from jax.experimental.pallas import tpu_sc as plsc

N = 10000
E = 320000
D = 128
N_PAD = 10240
NC = 2
NS = 16
L = 16
NW = NC * NS
K = 128
BPW = 80
EPW = BPW * K
E_PAD = NW * EPW
NBUF = 4
NGRP = BPW // NBUF
RSUB = N_PAD // NS
R = 1024
GRID = N_PAD // R

_MESH = plsc.VectorSubcoreMesh(core_axis_name="c", subcore_axis_name="s")


def _deg_body(dst_hbm, degp_hbm, dst_v, acc_v):
    c = lax.axis_index("c")
    s = lax.axis_index("s")
    wid = s * NC + c
    pltpu.sync_copy(dst_hbm.at[wid], dst_v)
    zeros = jnp.zeros((L,), jnp.float32)

    def zero_body(j, carry):
        acc_v[pl.ds(j * L, L)] = zeros
        return carry

    lax.fori_loop(0, N_PAD // L, zero_body, 0)
    ones = jnp.ones((L,), jnp.float32)

    def body(i, carry):
        idx = dst_v[pl.ds(i * L, L)]
        plsc.addupdate_scatter(acc_v, [idx], ones)
        return carry

    lax.fori_loop(0, EPW // L, body, 0)
    pltpu.sync_copy(acc_v, degp_hbm.at[wid])


_deg_call = pl.kernel(
    _deg_body,
    out_type=jax.ShapeDtypeStruct((NW, N_PAD), jnp.float32),
    mesh=_MESH,
    scratch_types=[
        pltpu.VMEM((EPW,), jnp.int32),
        pltpu.VMEM((N_PAD,), jnp.float32),
    ],
)


def _scatter_body(g_hbm, src_hbm, dst_hbm, out_hbm,
                  src_v, dst_v, rows_v, acc_sh, gsem, ssem):
    c = lax.axis_index("c")
    s = lax.axis_index("s")
    wid = s * NC + c
    pltpu.sync_copy(src_hbm.at[wid], src_v)
    pltpu.sync_copy(dst_hbm.at[wid], dst_v)

    zeros = jnp.zeros((L,), jnp.float32)
    for col in range(D // L):
        rows_v[0, 0, pl.ds(col * L, L)] = zeros
    filled = 1
    while filled < K:
        n = min(filled, K - filled)
        pltpu.sync_copy(rows_v.at[0, pl.ds(0, n)],
                        rows_v.at[0, pl.ds(filled, n)])
        filled += n
    base = s * RSUB
    for j in range(RSUB // K):
        pltpu.sync_copy(rows_v.at[0], acc_sh.at[pl.ds(base + j * K, K)])
    plsc.subcore_barrier()

    def group(grp, carry):
        gd = []
        for u in range(NBUF):
            b = grp * NBUF + u
            gd.append(pltpu.async_copy(
                g_hbm.at[src_v.at[b]], rows_v.at[u], gsem.at[u]))
        sd = []
        for u in range(NBUF):
            b = grp * NBUF + u
            gd[u].wait()
            sd.append(pltpu.async_copy(
                rows_v.at[u], acc_sh.at[dst_v.at[b]], ssem.at[u], add=True))
        for u in range(NBUF):
            sd[u].wait()
        return carry

    lax.fori_loop(0, NGRP, group, 0)
    plsc.subcore_barrier()
    pltpu.sync_copy(acc_sh.at[pl.ds(base, RSUB)],
                    out_hbm.at[c, pl.ds(base, RSUB)])


_scatter_call = pl.kernel(
    _scatter_body,
    out_type=jax.ShapeDtypeStruct((NC, N_PAD, D), jnp.float32),
    mesh=_MESH,
    scratch_types=[
        pltpu.VMEM((BPW, K), jnp.int32),
        pltpu.VMEM((BPW, K), jnp.int32),
        pltpu.VMEM((NBUF, K, D), jnp.float32),
        pltpu.VMEM_SHARED((N_PAD, D), jnp.float32),
        pltpu.SemaphoreType.DMA((NBUF,)),
        pltpu.SemaphoreType.DMA((NBUF,)),
    ],
)


def _tc1_body(x_ref, w_ref, degp_ref, g1_ref, dis_ref):
    ones = jnp.ones((NW, D), jnp.float32)
    deg = lax.dot_general(degp_ref[...], ones, (((0,), (0,)), ((), ())),
                          preferred_element_type=jnp.float32)
    dis = lax.rsqrt(deg + 1.0)
    h = jnp.dot(x_ref[...], w_ref[...], preferred_element_type=jnp.float32)
    g1_ref[...] = h * dis
    dis_ref[...] = dis


_tc1_call = pl.pallas_call(
    _tc1_body,
    grid=(GRID,),
    in_specs=[
        pl.BlockSpec((R, D), lambda i: (i, 0)),
        pl.BlockSpec((D, D), lambda i: (0, 0)),
        pl.BlockSpec((NW, R), lambda i: (0, i)),
    ],
    out_specs=[
        pl.BlockSpec((R, D), lambda i: (i, 0)),
        pl.BlockSpec((R, D), lambda i: (i, 0)),
    ],
    out_shape=[
        jax.ShapeDtypeStruct((N_PAD, D), jnp.float32),
        jax.ShapeDtypeStruct((N_PAD, D), jnp.float32),
    ],
)


def _tc2_body(accp_ref, g1_ref, dis_ref, b_ref, w_ref, g2_ref):
    agg = accp_ref[0] + accp_ref[1] + g1_ref[...]
    h1 = jnp.maximum(agg * dis_ref[...] + b_ref[...], 0.0)
    h = jnp.dot(h1, w_ref[...], preferred_element_type=jnp.float32)
    g2_ref[...] = h * dis_ref[...]


_tc2_call = pl.pallas_call(
    _tc2_body,
    grid=(GRID,),
    in_specs=[
        pl.BlockSpec((NC, R, D), lambda i: (0, i, 0)),
        pl.BlockSpec((R, D), lambda i: (i, 0)),
        pl.BlockSpec((R, D), lambda i: (i, 0)),
        pl.BlockSpec((D,), lambda i: (0,)),
        pl.BlockSpec((D, D), lambda i: (0, 0)),
    ],
    out_specs=pl.BlockSpec((R, D), lambda i: (i, 0)),
    out_shape=jax.ShapeDtypeStruct((N_PAD, D), jnp.float32),
)


def _tc3_body(accp_ref, g2_ref, dis_ref, b2_ref, w3_ref, b3_ref, out_ref):
    agg = accp_ref[0] + accp_ref[1] + g2_ref[...]
    h2 = jnp.maximum(agg * dis_ref[...] + b2_ref[...], 0.0)
    h = jnp.dot(h2, w3_ref[...], preferred_element_type=jnp.float32)
    out_ref[...] = jnp.maximum(h + b3_ref[...], 0.0)


_tc3_call = pl.pallas_call(
    _tc3_body,
    grid=(GRID,),
    in_specs=[
        pl.BlockSpec((NC, R, D), lambda i: (0, i, 0)),
        pl.BlockSpec((R, D), lambda i: (i, 0)),
        pl.BlockSpec((R, D), lambda i: (i, 0)),
        pl.BlockSpec((D,), lambda i: (0,)),
        pl.BlockSpec((D, D), lambda i: (0, 0)),
        pl.BlockSpec((D,), lambda i: (0,)),
    ],
    out_specs=pl.BlockSpec((R, D), lambda i: (i, 0)),
    out_shape=jax.ShapeDtypeStruct((N_PAD, D), jnp.float32),
)


@jax.jit
def kernel(x, edge_index, W1, b1, W2, b2, W3, b3):
    src = edge_index[0].astype(jnp.int32)
    dst = edge_index[1].astype(jnp.int32)
    pad = E_PAD - E
    src_p = jnp.concatenate(
        [src, jnp.zeros((pad,), jnp.int32)]).reshape(NW, BPW, K)
    dst_p = jnp.concatenate(
        [dst, N + (jnp.arange(pad, dtype=jnp.int32) % (N_PAD - N))]
    ).reshape(NW, BPW, K)
    x_p = jnp.pad(x, ((0, N_PAD - N), (0, 0)))

    degp = _deg_call(dst_p.reshape(NW, EPW))
    g1, dis = _tc1_call(x_p, W1, degp)
    acc1 = _scatter_call(g1, src_p, dst_p)
    g2 = _tc2_call(acc1, g1, dis, b1, W2)
    acc2 = _scatter_call(g2, src_p, dst_p)
    out = _tc3_call(acc2, g2, dis, b2, W3, b3)
    return out[:N]


# speedup vs baseline: 4.3570x; 4.3570x over previous
import jax
import jax.numpy as jnp
from jax import lax
from jax.experimental import pallas as pl
from jax.experimental.pallas import tpu as pltpu
from jax.experimental.pallas import tpu_sc as plsc

N = 10000
E = 320000
D = 128
N_PAD = 10240
NC = 2
NS = 16
L = 16
NW = NC * NS
K = 128
NWS = NS
NBUF = 2
NGRP = 81
BPWS = NGRP * NBUF
E_PAD = NWS * BPWS * K
EPW = E_PAD // NW
HALF = N_PAD // 2
DUMP = 512
ACC_ROWS = HALF + DUMP
ZBLK = ACC_ROWS // K
R = 1024
GRID = N_PAD // R

_MESH = plsc.VectorSubcoreMesh(core_axis_name="c", subcore_axis_name="s")


def _deg_body(dst_hbm, degp_hbm, dst_v, acc_v):
    c = lax.axis_index("c")
    s = lax.axis_index("s")
    wid = s * NC + c
    pltpu.sync_copy(dst_hbm.at[wid], dst_v)
    zeros = jnp.zeros((L,), jnp.float32)

    def zero_body(j, carry):
        acc_v[pl.ds(j * L, L)] = zeros
        return carry

    lax.fori_loop(0, N_PAD // L, zero_body, 0)
    ones = jnp.ones((L,), jnp.float32)

    def body(i, carry):
        idx = dst_v[pl.ds(i * L, L)]
        plsc.addupdate_scatter(acc_v, [idx], ones)
        return carry

    lax.fori_loop(0, EPW // L, body, 0)
    pltpu.sync_copy(acc_v, degp_hbm.at[wid])


_deg_call = pl.kernel(
    _deg_body,
    out_type=jax.ShapeDtypeStruct((NW, N_PAD), jnp.float32),
    mesh=_MESH,
    compiler_params=pltpu.CompilerParams(needs_layout_passes=False),
    scratch_types=[
        pltpu.VMEM((EPW,), jnp.int32),
        pltpu.VMEM((N_PAD,), jnp.float32),
    ],
)


def _scatter_body(g_hbm, edges_hbm, out_hbm,
                  idx_v, rows_v, acc_sh, gsem, ssem):
    c = lax.axis_index("c")
    s = lax.axis_index("s")
    pltpu.sync_copy(edges_hbm.at[s], idx_v)

    lo = c * HALF

    def loc_body(i, carry):
        b = i // (K // L)
        col = (i % (K // L)) * L
        d = idx_v[1, b, pl.ds(col, L)]
        dl = d - lo
        m = (dl >= 0) & (dl < HALF)
        dump = HALF + (d & (DUMP - 1))
        idx_v[1, b, pl.ds(col, L)] = jnp.where(m, dl, dump)
        return carry

    lax.fori_loop(0, BPWS * K // L, loc_body, 0)

    zeros = jnp.zeros((L,), jnp.float32)
    cpr = D // L

    def zero_body(i, carry):
        r = i // cpr
        col = (i % cpr) * L
        rows_v[0, r, pl.ds(col, L)] = zeros
        return carry

    lax.fori_loop(0, K * D // L, zero_body, 0)

    def zcopy_body(j, carry):
        blk = s + j * NS
        pltpu.sync_copy(rows_v.at[0], acc_sh.at[pl.ds(blk * K, K)])
        return carry

    lax.fori_loop(0, (ZBLK - s + NS - 1) // NS, zcopy_body, 0)
    plsc.subcore_barrier()

    def group(grp, carry):
        gd = []
        for u in range(NBUF):
            b = grp * NBUF + u
            gd.append(pltpu.async_copy(
                g_hbm.at[idx_v.at[0, b]], rows_v.at[u], gsem.at[u]))
        sd = []
        for u in range(NBUF):
            b = grp * NBUF + u
            gd[u].wait()
            sd.append(pltpu.async_copy(
                rows_v.at[u], acc_sh.at[idx_v.at[1, b]], ssem.at[u],
                add=True))
        for u in range(NBUF):
            sd[u].wait()
        return carry

    lax.fori_loop(0, NGRP, group, 0)
    plsc.subcore_barrier()
    ob = s * (HALF // NS)
    pltpu.sync_copy(acc_sh.at[pl.ds(ob, HALF // NS)],
                    out_hbm.at[c, pl.ds(ob, HALF // NS)])


_scatter_call = pl.kernel(
    _scatter_body,
    out_type=jax.ShapeDtypeStruct((NC, HALF, D), jnp.float32),
    mesh=_MESH,
    scratch_types=[
        pltpu.VMEM((2, BPWS, K), jnp.int32),
        pltpu.VMEM((NBUF, K, D), jnp.float32),
        pltpu.VMEM_SHARED((ACC_ROWS, D), jnp.float32),
        pltpu.SemaphoreType.DMA((NBUF,)),
        pltpu.SemaphoreType.DMA((NBUF,)),
    ],
)


def _tc1_body(x_ref, w_ref, degp_ref, g1_ref, dis_ref):
    ones = jnp.ones((NW, D), jnp.float32)
    deg = lax.dot_general(degp_ref[...], ones, (((0,), (0,)), ((), ())),
                          preferred_element_type=jnp.float32)
    dis = lax.rsqrt(deg + 1.0)
    h = jnp.dot(x_ref[...], w_ref[...], preferred_element_type=jnp.float32)
    g1_ref[...] = h * dis
    dis_ref[...] = dis


_tc1_call = pl.pallas_call(
    _tc1_body,
    grid=(GRID,),
    in_specs=[
        pl.BlockSpec((R, D), lambda i: (i, 0)),
        pl.BlockSpec((D, D), lambda i: (0, 0)),
        pl.BlockSpec((NW, R), lambda i: (0, i)),
    ],
    out_specs=[
        pl.BlockSpec((R, D), lambda i: (i, 0)),
        pl.BlockSpec((R, D), lambda i: (i, 0)),
    ],
    out_shape=[
        jax.ShapeDtypeStruct((N_PAD, D), jnp.float32),
        jax.ShapeDtypeStruct((N_PAD, D), jnp.float32),
    ],
)


def _tc2_body(acc_ref, g1_ref, dis_ref, b_ref, w_ref, g2_ref):
    agg = acc_ref[...] + g1_ref[...]
    h1 = jnp.maximum(agg * dis_ref[...] + b_ref[...], 0.0)
    h = jnp.dot(h1, w_ref[...], preferred_element_type=jnp.float32)
    g2_ref[...] = h * dis_ref[...]


_tc2_call = pl.pallas_call(
    _tc2_body,
    grid=(GRID,),
    in_specs=[
        pl.BlockSpec((R, D), lambda i: (i, 0)),
        pl.BlockSpec((R, D), lambda i: (i, 0)),
        pl.BlockSpec((R, D), lambda i: (i, 0)),
        pl.BlockSpec((D,), lambda i: (0,)),
        pl.BlockSpec((D, D), lambda i: (0, 0)),
    ],
    out_specs=pl.BlockSpec((R, D), lambda i: (i, 0)),
    out_shape=jax.ShapeDtypeStruct((N_PAD, D), jnp.float32),
)


def _tcf_body(g_ref, dis_ref, b3_ref, out_ref):
    h = g_ref[...] / dis_ref[...]
    out_ref[...] = jnp.maximum(h + b3_ref[...], 0.0)


_tcf_call = pl.pallas_call(
    _tcf_body,
    grid=(GRID,),
    in_specs=[
        pl.BlockSpec((R, D), lambda i: (i, 0)),
        pl.BlockSpec((R, D), lambda i: (i, 0)),
        pl.BlockSpec((D,), lambda i: (0,)),
    ],
    out_specs=pl.BlockSpec((R, D), lambda i: (i, 0)),
    out_shape=jax.ShapeDtypeStruct((N_PAD, D), jnp.float32),
)


@jax.jit
def kernel(x, edge_index, W1, b1, W2, b2, W3, b3):
    src = edge_index[0].astype(jnp.int32)
    dst = edge_index[1].astype(jnp.int32)
    pad = E_PAD - E
    src_flat = jnp.concatenate([src, jnp.zeros((pad,), jnp.int32)])
    dst_flat = jnp.concatenate(
        [dst, N + (jnp.arange(pad, dtype=jnp.int32) % (N_PAD - N))])
    edges = jnp.concatenate(
        [src_flat.reshape(NWS, 1, BPWS, K), dst_flat.reshape(NWS, 1, BPWS, K)],
        axis=1)
    x_p = jnp.pad(x, ((0, N_PAD - N), (0, 0)))

    degp = _deg_call(dst_flat.reshape(NW, EPW))
    g1, dis = _tc1_call(x_p, W1, degp)

    def conv_step(g, bw):
        b_i, w_i = bw
        acc = _scatter_call(g, edges).reshape(N_PAD, D)
        return _tc2_call(acc, g, dis, b_i, w_i), None

    g_fin, _ = lax.scan(conv_step, g1,
                        (jnp.stack([b1, b2]), jnp.stack([W2, W3])))
    out = _tcf_call(g_fin, dis, b3)
    return out[:N]


# speedup vs baseline: 5.8287x; 1.3378x over previous
import jax
import jax.numpy as jnp
from jax import lax
from jax.experimental import pallas as pl
from jax.experimental.pallas import tpu as pltpu
from jax.experimental.pallas import tpu_sc as plsc

N = 10000
E = 320000
D = 128
N_PAD = 10240
NC = 2
NS = 16
L = 16
NW = NC * NS
K = 128
NWS = NS
NBUF = 2
BPWS = 162
CAP = BPWS * K
GSZ = NBUF * K
E_PAD = NWS * BPWS * K
EPW = E_PAD // NW
HALF = N_PAD // 2
DUMP = 128
ACC_ROWS = HALF + DUMP
ZBLK = ACC_ROWS // K
R = 1024
GRID = N_PAD // R

_MESH = plsc.VectorSubcoreMesh(core_axis_name="c", subcore_axis_name="s")


def _deg_body(dst_hbm, degp_hbm, dst_v, acc_v):
    c = lax.axis_index("c")
    s = lax.axis_index("s")
    wid = s * NC + c
    pltpu.sync_copy(dst_hbm.at[wid], dst_v)
    zeros = jnp.zeros((L,), jnp.float32)

    def zero_body(j, carry):
        acc_v[pl.ds(j * L, L)] = zeros
        return carry

    lax.fori_loop(0, N_PAD // L, zero_body, 0)
    ones = jnp.ones((L,), jnp.float32)

    def body(i, carry):
        idx = dst_v[pl.ds(i * L, L)]
        plsc.addupdate_scatter(acc_v, [idx], ones)
        return carry

    lax.fori_loop(0, EPW // L, body, 0)
    pltpu.sync_copy(acc_v, degp_hbm.at[wid])


_deg_call = pl.kernel(
    _deg_body,
    out_type=jax.ShapeDtypeStruct((NW, N_PAD), jnp.float32),
    mesh=_MESH,
    compiler_params=pltpu.CompilerParams(needs_layout_passes=False),
    scratch_types=[
        pltpu.VMEM((EPW,), jnp.int32),
        pltpu.VMEM((N_PAD,), jnp.float32),
    ],
)


def _scatter_body(g_hbm, edges_hbm, out_hbm,
                  src_c, dst_c, srcb, dstb, rows_v, acc_sh, gsem, ssem):
    c = lax.axis_index("c")
    s = lax.axis_index("s")
    pltpu.sync_copy(edges_hbm.at[s, 0], src_c.at[pl.ds(0, CAP)])
    pltpu.sync_copy(edges_hbm.at[s, 1], dst_c.at[pl.ds(0, CAP)])

    lo = c * HALF
    lane = lax.broadcasted_iota(jnp.int32, (L,), 0)

    def comp_body(i, o):
        sv = src_c[pl.ds(i * L, L)]
        dv = dst_c[pl.ds(i * L, L)]
        dl = dv - lo
        m = (dl >= 0) & (dl < HALF)
        cm = plsc.cumsum(m.astype(jnp.int32))
        pos = jnp.where(m, o + cm - 1, CAP + lane)
        plsc.store_scatter(src_c, [pos], sv)
        plsc.store_scatter(dst_c, [pos], dl)
        return o + lax.reduce_max(cm, (0,))

    o = lax.fori_loop(0, CAP // L, comp_body, jnp.int32(0))

    ngrp = (o + GSZ - 1) // GSZ
    end = ngrp * GSZ
    base0 = (o // L) * L

    @pl.when(o < end)
    def _pad_partial():
        ix = jnp.minimum(o + lane, end - 1)
        plsc.store_scatter(src_c, [ix], jnp.zeros((L,), jnp.int32))
        plsc.store_scatter(dst_c, [ix], HALF + (ix & (DUMP - 1)))

    def pad_body(j, carry):
        b0 = j * L
        src_c[pl.ds(b0, L)] = jnp.zeros((L,), jnp.int32)
        dst_c[pl.ds(b0, L)] = HALF + ((b0 + lane) & (DUMP - 1))
        return carry

    lax.fori_loop(o // L + 1, end // L, pad_body, 0)

    zeros = jnp.zeros((L,), jnp.float32)
    cpr = D // L

    def zero_body(i, carry):
        r = i // cpr
        col = (i % cpr) * L
        rows_v[0, r, pl.ds(col, L)] = zeros
        return carry

    lax.fori_loop(0, K * D // L, zero_body, 0)

    def zcopy_body(j, carry):
        blk = s + j * NS
        pltpu.sync_copy(rows_v.at[0], acc_sh.at[pl.ds(blk * K, K)])
        return carry

    lax.fori_loop(0, (ZBLK - s + NS - 1) // NS, zcopy_body, 0)
    plsc.subcore_barrier()

    def group(grp, carry):
        for u in range(NBUF):
            b = grp * NBUF + u
            for col in range(K // L):
                srcb[u, pl.ds(col * L, L)] = src_c[pl.ds(b * K + col * L, L)]
                dstb[u, pl.ds(col * L, L)] = dst_c[pl.ds(b * K + col * L, L)]
        gd = []
        for u in range(NBUF):
            gd.append(pltpu.async_copy(
                g_hbm.at[srcb.at[u]], rows_v.at[u], gsem.at[u]))
        sd = []
        for u in range(NBUF):
            gd[u].wait()
            sd.append(pltpu.async_copy(
                rows_v.at[u], acc_sh.at[dstb.at[u]], ssem.at[u],
                add=True))
        for u in range(NBUF):
            sd[u].wait()
        return carry

    lax.fori_loop(0, ngrp, group, 0)
    plsc.subcore_barrier()
    ob = s * (HALF // NS)
    pltpu.sync_copy(acc_sh.at[pl.ds(ob, HALF // NS)],
                    out_hbm.at[c, pl.ds(ob, HALF // NS)])


_scatter_call = pl.kernel(
    _scatter_body,
    out_type=jax.ShapeDtypeStruct((NC, HALF, D), jnp.float32),
    mesh=_MESH,
    compiler_params=pltpu.CompilerParams(needs_layout_passes=False),
    scratch_types=[
        pltpu.VMEM((CAP + L,), jnp.int32),
        pltpu.VMEM((CAP + L,), jnp.int32),
        pltpu.VMEM((NBUF, K), jnp.int32),
        pltpu.VMEM((NBUF, K), jnp.int32),
        pltpu.VMEM((NBUF, K, D), jnp.float32),
        pltpu.VMEM_SHARED((ACC_ROWS, D), jnp.float32),
        pltpu.SemaphoreType.DMA((NBUF,)),
        pltpu.SemaphoreType.DMA((NBUF,)),
    ],
)


def _tc1_body(x_ref, w_ref, degp_ref, g1_ref, dis_ref):
    ones = jnp.ones((NW, D), jnp.float32)
    deg = lax.dot_general(degp_ref[...], ones, (((0,), (0,)), ((), ())),
                          preferred_element_type=jnp.float32)
    dis = lax.rsqrt(deg + 1.0)
    h = jnp.dot(x_ref[...], w_ref[...], preferred_element_type=jnp.float32)
    g1_ref[...] = h * dis
    dis_ref[...] = dis


_tc1_call = pl.pallas_call(
    _tc1_body,
    grid=(GRID,),
    in_specs=[
        pl.BlockSpec((R, D), lambda i: (i, 0)),
        pl.BlockSpec((D, D), lambda i: (0, 0)),
        pl.BlockSpec((NW, R), lambda i: (0, i)),
    ],
    out_specs=[
        pl.BlockSpec((R, D), lambda i: (i, 0)),
        pl.BlockSpec((R, D), lambda i: (i, 0)),
    ],
    out_shape=[
        jax.ShapeDtypeStruct((N_PAD, D), jnp.float32),
        jax.ShapeDtypeStruct((N_PAD, D), jnp.float32),
    ],
)


def _tc2_body(acc_ref, g1_ref, dis_ref, b_ref, w_ref, g2_ref):
    agg = acc_ref[...] + g1_ref[...]
    h1 = jnp.maximum(agg * dis_ref[...] + b_ref[...], 0.0)
    h = jnp.dot(h1, w_ref[...], preferred_element_type=jnp.float32)
    g2_ref[...] = h * dis_ref[...]


_tc2_call = pl.pallas_call(
    _tc2_body,
    grid=(GRID,),
    in_specs=[
        pl.BlockSpec((R, D), lambda i: (i, 0)),
        pl.BlockSpec((R, D), lambda i: (i, 0)),
        pl.BlockSpec((R, D), lambda i: (i, 0)),
        pl.BlockSpec((D,), lambda i: (0,)),
        pl.BlockSpec((D, D), lambda i: (0, 0)),
    ],
    out_specs=pl.BlockSpec((R, D), lambda i: (i, 0)),
    out_shape=jax.ShapeDtypeStruct((N_PAD, D), jnp.float32),
)


def _tcf_body(g_ref, dis_ref, b3_ref, out_ref):
    h = g_ref[...] / dis_ref[...]
    out_ref[...] = jnp.maximum(h + b3_ref[...], 0.0)


_tcf_call = pl.pallas_call(
    _tcf_body,
    grid=(GRID,),
    in_specs=[
        pl.BlockSpec((R, D), lambda i: (i, 0)),
        pl.BlockSpec((R, D), lambda i: (i, 0)),
        pl.BlockSpec((D,), lambda i: (0,)),
    ],
    out_specs=pl.BlockSpec((R, D), lambda i: (i, 0)),
    out_shape=jax.ShapeDtypeStruct((N_PAD, D), jnp.float32),
)


@jax.jit
def kernel(x, edge_index, W1, b1, W2, b2, W3, b3):
    src = edge_index[0].astype(jnp.int32)
    dst = edge_index[1].astype(jnp.int32)
    pad = E_PAD - E
    src_flat = jnp.concatenate([src, jnp.zeros((pad,), jnp.int32)])
    dst_flat = jnp.concatenate(
        [dst, N + (jnp.arange(pad, dtype=jnp.int32) % (N_PAD - N))])
    edges = jnp.concatenate(
        [src_flat.reshape(NWS, 1, CAP), dst_flat.reshape(NWS, 1, CAP)],
        axis=1)
    x_p = jnp.pad(x, ((0, N_PAD - N), (0, 0)))

    degp = _deg_call(dst_flat.reshape(NW, EPW))
    g1, dis = _tc1_call(x_p, W1, degp)

    def conv_step(g, bw):
        b_i, w_i = bw
        acc = _scatter_call(g, edges).reshape(N_PAD, D)
        return _tc2_call(acc, g, dis, b_i, w_i), None

    g_fin, _ = lax.scan(conv_step, g1,
                        (jnp.stack([b1, b2]), jnp.stack([W2, W3])))
    out = _tcf_call(g_fin, dis, b3)
    return out[:N]
